# baseline (speedup 1.0000x reference)
import functools

import numpy as np
import jax
import jax.numpy as jnp
from jax import lax
from jax.experimental import pallas as pl
from jax.experimental.pallas import tpu as pltpu
from jax.experimental.pallas import tpu_sc as plsc

B, C, H, W = 4, 19, 512, 512
N = B * H * W
N_MIN = N // 16
THRESH = float(-np.log(0.7))

HT = 256
HS = H - HT
N_TC = B * HT * W
N_SC = B * HS * W

BH = 32

_NC, _NS, _L = 2, 16, 16
_NW = _NC * _NS
_CHUNK = N // _NW

_TECS_PER_B = _NW // B
_PX = HS * W // _TECS_PER_B
_NROWS = _PX // W

_LOG_POLY = [3.3423269485410856e-08, 1.0000030994415283, -0.5000129342079163,
             0.33304813504219055, -0.2491120994091034, 0.20611785352230072,
             -0.18627697229385376, 0.11448435485363007]
_SQRT2 = float(np.sqrt(2.0))
_LN2 = float(np.log(2.0))


def _i32(x):
    return jnp.int32(np.int32(np.uint32(x)))


def _loss_body(pred_ref, tgt_ref, loss_ref, stat_ref):
    m = pred_ref[0, 0]
    for c in range(1, C):
        m = jnp.maximum(m, pred_ref[0, c])
    s = jnp.exp(pred_ref[0, 0] - m)
    for c in range(1, C):
        s = s + jnp.exp(pred_ref[0, c] - m)
    tb = tgt_ref[0, 0]
    pb = pred_ref[0, 0]
    for c in range(1, C):
        tc = tgt_ref[0, c]
        upd = tc > tb
        tb = jnp.where(upd, tc, tb)
        pb = jnp.where(upd, pred_ref[0, c], pb)
    loss = jnp.maximum((m + jnp.log(s)) - pb, 0.0)
    loss_ref[0] = loss

    hard = loss > THRESH
    cnt = jnp.where(hard, 1.0, 0.0)
    sm = jnp.where(hard, loss, 0.0)
    cpart = jnp.zeros((8, 128), jnp.float32)
    spart = jnp.zeros((8, 128), jnp.float32)
    for r in range(BH // 8):
        for c2 in range(W // 128):
            cpart = cpart + cnt[r * 8:(r + 1) * 8, c2 * 128:(c2 + 1) * 128]
            spart = spart + sm[r * 8:(r + 1) * 8, c2 * 128:(c2 + 1) * 128]

    @pl.when((pl.program_id(0) + pl.program_id(1)) == 0)
    def _():
        stat_ref[...] = jnp.zeros_like(stat_ref)

    stat_ref[0] += cpart
    stat_ref[1] += spart


def _loss_pallas(pred, targets):
    return pl.pallas_call(
        _loss_body,
        grid=(B, HT // BH),
        in_specs=[
            pl.BlockSpec((1, C, BH, W), lambda b, i: (b, 0, i, 0)),
            pl.BlockSpec((1, C, BH, W), lambda b, i: (b, 0, i, 0)),
        ],
        out_specs=[
            pl.BlockSpec((1, BH, W), lambda b, i: (b, i, 0)),
            pl.BlockSpec((2, 8, 128), lambda b, i: (0, 0, 0)),
        ],
        out_shape=[
            jax.ShapeDtypeStruct((B, HT, W), jnp.float32),
            jax.ShapeDtypeStruct((2, 8, 128), jnp.float32),
        ],
    )(pred, targets)


def _sc_loss_body(pred_hbm, tgt_hbm, loss_out, sum_out, cnt_out,
                  pred_v, tgt_v, loss_v, sum_v, cnt_v,
                  sem_in0, sem_in1):
    wid = lax.axis_index("s") * _NC + lax.axis_index("c")
    b = wid // _TECS_PER_B
    t = wid % _TECS_PER_B
    row0 = HT + t * _NROWS
    obase = b * (HS * W) + t * _PX

    in_sems = [sem_in0, sem_in1]

    def issue(slot, j):
        po = (row0 + j) * W
        pltpu.async_copy(pred_hbm.at[pl.ds(b, 1), :, pl.ds(po, W)],
                         pred_v.at[slot], in_sems[slot])
        pltpu.async_copy(tgt_hbm.at[pl.ds(b, 1), :, pl.ds(po, W)],
                         tgt_v.at[slot], in_sems[slot])

    def wait_in(slot, j):
        po = (row0 + j) * W
        pltpu.make_async_copy(pred_hbm.at[pl.ds(b, 1), :, pl.ds(po, W)],
                              pred_v.at[slot], in_sems[slot]).wait()
        pltpu.make_async_copy(tgt_hbm.at[pl.ds(b, 1), :, pl.ds(po, W)],
                              tgt_v.at[slot], in_sems[slot]).wait()

    def flush_out(slot, j):
        pltpu.sync_copy(loss_v.at[slot],
                        loss_out.at[pl.ds(obase + j * W, W)])

    thr = jnp.float32(THRESH)
    sqrt2 = jnp.float32(_SQRT2)
    ln2 = jnp.float32(_LN2)

    def compute(slot, carry):
        def grp(g, carry):
            ca, sa = carry
            ds = pl.ds(g * _L, _L)
            ps = [pred_v[slot, 0, c, ds] for c in range(C)]
            m = ps[0]
            tb = tgt_v[slot, 0, 0, ds]
            pb = ps[0]
            for c in range(1, C):
                tc = tgt_v[slot, 0, c, ds]
                m = jnp.maximum(m, ps[c])
                upd = tc > tb
                tb = jnp.where(upd, tc, tb)
                pb = jnp.where(upd, ps[c], pb)
            s = jnp.exp(ps[0] - m)
            for c in range(1, C):
                s = s + jnp.exp(ps[c] - m)
            bits = lax.bitcast_convert_type(s, jnp.int32)
            e = lax.shift_right_logical(bits, 23) - 127
            f = lax.bitcast_convert_type(
                (bits & _i32(0x007FFFFF)) | _i32(0x3F800000), jnp.float32)
            red = f >= sqrt2
            f = jnp.where(red, f * 0.5, f)
            e = e + jnp.where(red, 1, 0)
            z = f - 1.0
            acc = jnp.float32(_LOG_POLY[-1])
            for coef in _LOG_POLY[-2::-1]:
                acc = acc * z + jnp.float32(coef)
            lg = e.astype(jnp.float32) * ln2 + acc
            loss = jnp.maximum((m + lg) - pb, 0.0)
            loss_v[slot, ds] = loss
            hard = loss > thr
            ca = ca + jnp.where(hard, 1.0, 0.0)
            sa = sa + jnp.where(hard, loss, 0.0)
            return ca, sa

        return lax.fori_loop(0, W // _L, grp, carry)

    carry0 = (jnp.zeros((_L,), jnp.float32), jnp.zeros((_L,), jnp.float32))
    issue(0, 0)

    def pair_body(jj, carry):
        j0 = 2 * jj
        issue(1, j0 + 1)
        wait_in(0, j0)
        carry = compute(0, carry)
        flush_out(0, j0)
        issue(0, j0 + 2)
        wait_in(1, j0 + 1)
        carry = compute(1, carry)
        flush_out(1, j0 + 1)
        return carry

    carry = lax.fori_loop(0, _NROWS // 2 - 1, pair_body, carry0)
    jl = _NROWS - 2
    issue(1, jl + 1)
    wait_in(0, jl)
    carry = compute(0, carry)
    flush_out(0, jl)
    wait_in(1, jl + 1)
    carry = compute(1, carry)
    flush_out(1, jl + 1)

    cnt_v[...] = carry[0]
    sum_v[...] = carry[1]
    pltpu.sync_copy(sum_v, sum_out.at[wid])
    pltpu.sync_copy(cnt_v, cnt_out.at[wid])


@functools.lru_cache
def _get_sc_loss_kernel():
    mesh = plsc.VectorSubcoreMesh(
        core_axis_name="c", subcore_axis_name="s",
        num_cores=_NC, num_subcores=_NS)
    return functools.partial(
        pl.kernel,
        mesh=mesh,
        out_type=[
            jax.ShapeDtypeStruct((N_SC,), jnp.float32),
            jax.ShapeDtypeStruct((_NW, _L), jnp.float32),
            jax.ShapeDtypeStruct((_NW, _L), jnp.float32),
        ],
        scratch_types=[
            pltpu.VMEM((2, 1, C, W), jnp.float32),
            pltpu.VMEM((2, 1, C, W), jnp.float32),
            pltpu.VMEM((2, W), jnp.float32),
            pltpu.VMEM((_L,), jnp.float32),
            pltpu.VMEM((_L,), jnp.float32),
            pltpu.SemaphoreType.DMA,
            pltpu.SemaphoreType.DMA,
        ],
        compiler_params=pltpu.CompilerParams(needs_layout_passes=False),
    )(_sc_loss_body)


@functools.lru_cache
def _make_sc_hist(shift, mask, nbins):
    mesh = plsc.VectorSubcoreMesh(
        core_axis_name="c", subcore_axis_name="s",
        num_cores=_NC, num_subcores=_NS)

    @functools.partial(
        pl.kernel,
        mesh=mesh,
        out_type=[
            jax.ShapeDtypeStruct((_NW, nbins), jnp.int32),
            jax.ShapeDtypeStruct((_NW, nbins), jnp.float32),
        ],
        scratch_types=[
            pltpu.VMEM((_CHUNK,), jnp.float32),
            pltpu.VMEM((_L,), jnp.int32),
            pltpu.VMEM((_L * nbins,), jnp.int32),
            pltpu.VMEM((_L * nbins,), jnp.float32),
            pltpu.VMEM((nbins,), jnp.int32),
            pltpu.VMEM((nbins,), jnp.float32),
        ],
        compiler_params=pltpu.CompilerParams(needs_layout_passes=False),
    )
    def _sc_hist(loss_hbm, prefix_hbm, cnt_out, sum_out,
                 data_v, pref_v, histc_v, hists_v, outc_v, outs_v):
        wid = lax.axis_index("s") * _NC + lax.axis_index("c")
        base = wid * _CHUNK
        pltpu.sync_copy(loss_hbm.at[pl.ds(base, _CHUNK)], data_v)
        pltpu.sync_copy(prefix_hbm, pref_v)

        zi = jnp.zeros((_L,), jnp.int32)
        zf = jnp.zeros((_L,), jnp.float32)

        def zero_body(i, _):
            histc_v[pl.ds(i * _L, _L)] = zi
            hists_v[pl.ds(i * _L, _L)] = zf
            return 0

        lax.fori_loop(0, nbins, zero_body, 0)

        pref = pref_v[...]
        lane_off = lax.iota(jnp.int32, _L) * nbins
        ones = jnp.ones((_L,), jnp.int32)
        maskc = _i32(mask)
        binm = _i32(nbins - 1)

        def body(i, _):
            v = data_v[pl.ds(i * _L, _L)]
            bits = lax.bitcast_convert_type(v, jnp.int32)
            match = (bits & maskc) == pref
            bn = lax.shift_right_logical(bits, shift) & binm
            idx = lane_off + bn
            plsc.addupdate_scatter(histc_v, [idx], ones, mask=match)
            plsc.addupdate_scatter(hists_v, [idx], v, mask=match)
            return 0

        lax.fori_loop(0, _CHUNK // _L, body, 0)

        def red_body(g, _):
            accc = zi
            accs = zf
            for l in range(_L):
                accc = accc + histc_v[pl.ds(l * nbins + g * _L, _L)]
                accs = accs + hists_v[pl.ds(l * nbins + g * _L, _L)]
            outc_v[pl.ds(g * _L, _L)] = accc
            outs_v[pl.ds(g * _L, _L)] = accs
            return 0

        lax.fori_loop(0, nbins // _L, red_body, 0)
        pltpu.sync_copy(outc_v, cnt_out.at[wid])
        pltpu.sync_copy(outs_v, sum_out.at[wid])

    return _sc_hist


def _select_level(counts, sums, k_rem):
    cum_ge_c = jnp.cumsum(counts[::-1])[::-1]
    cum_ge_s = jnp.cumsum(sums[::-1])[::-1]
    idx = jnp.arange(counts.shape[0], dtype=jnp.int32)
    b = jnp.max(jnp.where(cum_ge_c >= k_rem, idx, -1))
    cnt_above = cum_ge_c[b] - counts[b]
    sum_above = cum_ge_s[b] - sums[b]
    return b, cnt_above, sum_above, k_rem - cnt_above


def _topk_mean(loss_flat, k):
    kf = k.astype(jnp.float32)

    def level(hist_fn, prefix_bits):
        pref = jnp.broadcast_to(prefix_bits.astype(jnp.int32), (_L,))
        cnt_p, sum_p = hist_fn(loss_flat, pref)
        return cnt_p.sum(axis=0), sum_p.sum(axis=0)

    c1, s1 = level(_make_sc_hist(21, 0x00000000, 2048), jnp.int32(0))
    b1, ca1, sa1, k1 = _select_level(c1, s1, k)
    pref2 = lax.shift_left(b1, 21)
    c2, s2 = level(_make_sc_hist(10, 0xFFE00000, 2048), pref2)
    b2, ca2, sa2, k2 = _select_level(c2, s2, k1)
    pref3 = pref2 | lax.shift_left(b2, 10)
    c3, s3 = level(_make_sc_hist(0, 0xFFFFFC00, 1024), pref3)
    b3, ca3, sa3, k3 = _select_level(c3, s3, k2)
    v_bits = pref3 | b3
    v = lax.bitcast_convert_type(v_bits, jnp.float32)
    count_gt = ca1 + ca2 + ca3
    sum_gt = sa1 + sa2 + sa3
    return (sum_gt + (k - count_gt).astype(jnp.float32) * v) / kf


def kernel(pred, targets):
    loss_tc, stat_tc = _loss_pallas(pred, targets)
    loss_sc, sum_sc, cnt_sc = _get_sc_loss_kernel()(
        pred.reshape(B, C, H * W), targets.reshape(B, C, H * W))
    count_hard = jnp.sum(stat_tc[0]) + jnp.sum(cnt_sc)
    sum_hard = jnp.sum(stat_tc[1]) + jnp.sum(sum_sc)
    return lax.cond(
        count_hard >= N_MIN,
        lambda args: args[2] / args[3],
        lambda args: _topk_mean(
            jnp.concatenate([args[0].reshape(-1), args[1]]),
            jnp.int32(N_MIN)),
        (loss_tc, loss_sc, sum_hard, count_hard),
    )


# speedup vs baseline: 2.9397x; 2.9397x over previous
import functools

import numpy as np
import jax
import jax.numpy as jnp
from jax import lax
from jax.experimental import pallas as pl
from jax.experimental.pallas import tpu as pltpu
from jax.experimental.pallas import tpu_sc as plsc

B, C, H, W = 4, 19, 512, 512
N = B * H * W
N_MIN = N // 16
THRESH = float(-np.log(0.7))

BH = 32

_NC, _NS, _L = 2, 16, 16
_NW = _NC * _NS
_CHUNK = N // _NW


def _i32(x):
    return jnp.int32(np.int32(np.uint32(x)))


def _loss_body(pred_ref, tgt_ref, loss_ref):
    m = pred_ref[0, 0]
    for c in range(1, C):
        m = jnp.maximum(m, pred_ref[0, c])
    s = jnp.exp(pred_ref[0, 0] - m)
    for c in range(1, C):
        s = s + jnp.exp(pred_ref[0, c] - m)
    tb = tgt_ref[0, 0]
    pb = pred_ref[0, 0]
    for c in range(1, C):
        tc = tgt_ref[0, c]
        upd = tc > tb
        tb = jnp.where(upd, tc, tb)
        pb = jnp.where(upd, pred_ref[0, c], pb)
    loss_ref[0] = jnp.maximum((m + jnp.log(s)) - pb, 0.0)


def _loss_pallas(pred, targets):
    return pl.pallas_call(
        _loss_body,
        grid=(B, H // BH),
        in_specs=[
            pl.BlockSpec((1, C, BH, W), lambda b, i: (b, 0, i, 0)),
            pl.BlockSpec((1, C, BH, W), lambda b, i: (b, 0, i, 0)),
        ],
        out_specs=pl.BlockSpec((1, BH, W), lambda b, i: (b, i, 0)),
        out_shape=jax.ShapeDtypeStruct((B, H, W), jnp.float32),
    )(pred, targets)


def _sc_hard_stats_body(loss_hbm, sum_out, cnt_out, data_v, sum_v, cnt_v):
    wid = lax.axis_index("s") * _NC + lax.axis_index("c")
    base = wid * _CHUNK
    pltpu.sync_copy(loss_hbm.at[pl.ds(base, _CHUNK)], data_v)

    thr = jnp.float32(THRESH)

    def body(i, carry):
        sacc, cacc = carry
        for u in range(4):
            v = data_v[pl.ds((i * 4 + u) * _L, _L)]
            hard = v > thr
            sacc = sacc + jnp.where(hard, v, jnp.float32(0.0))
            cacc = cacc + jnp.where(hard, jnp.float32(1.0), jnp.float32(0.0))
        return sacc, cacc

    sacc, cacc = lax.fori_loop(
        0, _CHUNK // (_L * 4), body,
        (jnp.zeros((_L,), jnp.float32), jnp.zeros((_L,), jnp.float32)),
    )
    sum_v[...] = sacc
    cnt_v[...] = cacc
    pltpu.sync_copy(sum_v, sum_out.at[wid])
    pltpu.sync_copy(cnt_v, cnt_out.at[wid])


@functools.lru_cache
def _get_hard_stats_kernel():
    mesh = plsc.VectorSubcoreMesh(
        core_axis_name="c", subcore_axis_name="s",
        num_cores=_NC, num_subcores=_NS)
    return functools.partial(
        pl.kernel,
        mesh=mesh,
        out_type=[
            jax.ShapeDtypeStruct((_NW, _L), jnp.float32),
            jax.ShapeDtypeStruct((_NW, _L), jnp.float32),
        ],
        scratch_types=[
            pltpu.VMEM((_CHUNK,), jnp.float32),
            pltpu.VMEM((_L,), jnp.float32),
            pltpu.VMEM((_L,), jnp.float32),
        ],
        compiler_params=pltpu.CompilerParams(needs_layout_passes=False),
    )(_sc_hard_stats_body)


@functools.lru_cache
def _make_sc_hist(shift, mask, nbins):
    mesh = plsc.VectorSubcoreMesh(
        core_axis_name="c", subcore_axis_name="s",
        num_cores=_NC, num_subcores=_NS)

    @functools.partial(
        pl.kernel,
        mesh=mesh,
        out_type=[
            jax.ShapeDtypeStruct((_NW, nbins), jnp.int32),
            jax.ShapeDtypeStruct((_NW, nbins), jnp.float32),
        ],
        scratch_types=[
            pltpu.VMEM((_CHUNK,), jnp.float32),
            pltpu.VMEM((_L,), jnp.int32),
            pltpu.VMEM((_L * nbins,), jnp.int32),
            pltpu.VMEM((_L * nbins,), jnp.float32),
            pltpu.VMEM((nbins,), jnp.int32),
            pltpu.VMEM((nbins,), jnp.float32),
        ],
        compiler_params=pltpu.CompilerParams(needs_layout_passes=False),
    )
    def _sc_hist(loss_hbm, prefix_hbm, cnt_out, sum_out,
                 data_v, pref_v, histc_v, hists_v, outc_v, outs_v):
        wid = lax.axis_index("s") * _NC + lax.axis_index("c")
        base = wid * _CHUNK
        pltpu.sync_copy(loss_hbm.at[pl.ds(base, _CHUNK)], data_v)
        pltpu.sync_copy(prefix_hbm, pref_v)

        zi = jnp.zeros((_L,), jnp.int32)
        zf = jnp.zeros((_L,), jnp.float32)

        def zero_body(i, _):
            histc_v[pl.ds(i * _L, _L)] = zi
            hists_v[pl.ds(i * _L, _L)] = zf
            return 0

        lax.fori_loop(0, nbins, zero_body, 0)

        pref = pref_v[...]
        lane_off = lax.iota(jnp.int32, _L) * nbins
        ones = jnp.ones((_L,), jnp.int32)
        maskc = _i32(mask)
        binm = _i32(nbins - 1)

        def body(i, _):
            v = data_v[pl.ds(i * _L, _L)]
            bits = lax.bitcast_convert_type(v, jnp.int32)
            match = (bits & maskc) == pref
            bn = lax.shift_right_logical(bits, shift) & binm
            idx = lane_off + bn
            plsc.addupdate_scatter(histc_v, [idx], ones, mask=match)
            plsc.addupdate_scatter(hists_v, [idx], v, mask=match)
            return 0

        lax.fori_loop(0, _CHUNK // _L, body, 0)

        def red_body(g, _):
            accc = zi
            accs = zf
            for l in range(_L):
                accc = accc + histc_v[pl.ds(l * nbins + g * _L, _L)]
                accs = accs + hists_v[pl.ds(l * nbins + g * _L, _L)]
            outc_v[pl.ds(g * _L, _L)] = accc
            outs_v[pl.ds(g * _L, _L)] = accs
            return 0

        lax.fori_loop(0, nbins // _L, red_body, 0)
        pltpu.sync_copy(outc_v, cnt_out.at[wid])
        pltpu.sync_copy(outs_v, sum_out.at[wid])

    return _sc_hist


def _select_level(counts, sums, k_rem):
    cum_ge_c = jnp.cumsum(counts[::-1])[::-1]
    cum_ge_s = jnp.cumsum(sums[::-1])[::-1]
    idx = jnp.arange(counts.shape[0], dtype=jnp.int32)
    b = jnp.max(jnp.where(cum_ge_c >= k_rem, idx, -1))
    cnt_above = cum_ge_c[b] - counts[b]
    sum_above = cum_ge_s[b] - sums[b]
    return b, cnt_above, sum_above, k_rem - cnt_above


def _topk_mean(loss_flat, k):
    kf = k.astype(jnp.float32)

    def level(hist_fn, prefix_bits):
        pref = jnp.broadcast_to(prefix_bits.astype(jnp.int32), (_L,))
        cnt_p, sum_p = hist_fn(loss_flat, pref)
        return cnt_p.sum(axis=0), sum_p.sum(axis=0)

    c1, s1 = level(_make_sc_hist(21, 0x00000000, 2048), jnp.int32(0))
    b1, ca1, sa1, k1 = _select_level(c1, s1, k)
    pref2 = lax.shift_left(b1, 21)
    c2, s2 = level(_make_sc_hist(10, 0xFFE00000, 2048), pref2)
    b2, ca2, sa2, k2 = _select_level(c2, s2, k1)
    pref3 = pref2 | lax.shift_left(b2, 10)
    c3, s3 = level(_make_sc_hist(0, 0xFFFFFC00, 1024), pref3)
    b3, ca3, sa3, k3 = _select_level(c3, s3, k2)
    v_bits = pref3 | b3
    v = lax.bitcast_convert_type(v_bits, jnp.float32)
    count_gt = ca1 + ca2 + ca3
    sum_gt = sa1 + sa2 + sa3
    return (sum_gt + (k - count_gt).astype(jnp.float32) * v) / kf


def kernel(pred, targets):
    loss_flat = _loss_pallas(pred, targets).reshape(N)
    sum_p, cnt_p = _get_hard_stats_kernel()(loss_flat)
    sum_hard = jnp.sum(sum_p)
    count_hard = jnp.sum(cnt_p)
    return lax.cond(
        count_hard >= N_MIN,
        lambda args: args[1] / args[2],
        lambda args: _topk_mean(args[0], jnp.int32(N_MIN)),
        (loss_flat, sum_hard, count_hard),
    )


# speedup vs baseline: 3.5198x; 1.1973x over previous
import functools

import numpy as np
import jax
import jax.numpy as jnp
from jax import lax
from jax.experimental import pallas as pl
from jax.experimental.pallas import tpu as pltpu
from jax.experimental.pallas import tpu_sc as plsc

B, C, H, W = 4, 19, 512, 512
N = B * H * W
N_MIN = N // 16
THRESH = float(-np.log(0.7))

BH = 64

_NC, _NS, _L = 2, 16, 16
_NW = _NC * _NS
_CHUNK = N // _NW


def _i32(x):
    return jnp.int32(np.int32(np.uint32(x)))


def _loss_body(pred_ref, tgt_ref, loss_ref):
    m = pred_ref[0, 0]
    for c in range(1, C):
        m = jnp.maximum(m, pred_ref[0, c])
    s = jnp.exp(pred_ref[0, 0] - m)
    for c in range(1, C):
        s = s + jnp.exp(pred_ref[0, c] - m)
    tb = tgt_ref[0, 0]
    pb = pred_ref[0, 0]
    for c in range(1, C):
        tc = tgt_ref[0, c]
        upd = tc > tb
        tb = jnp.where(upd, tc, tb)
        pb = jnp.where(upd, pred_ref[0, c], pb)
    loss_ref[0] = jnp.maximum((m + jnp.log(s)) - pb, 0.0)


def _loss_pallas(pred, targets):
    return pl.pallas_call(
        _loss_body,
        grid=(B, H // BH),
        in_specs=[
            pl.BlockSpec((1, C, BH, W), lambda b, i: (b, 0, i, 0)),
            pl.BlockSpec((1, C, BH, W), lambda b, i: (b, 0, i, 0)),
        ],
        out_specs=pl.BlockSpec((1, BH, W), lambda b, i: (b, i, 0)),
        out_shape=jax.ShapeDtypeStruct((B, H, W), jnp.float32),
    )(pred, targets)


def _sc_hard_stats_body(loss_hbm, sum_out, cnt_out, data_v, sum_v, cnt_v):
    wid = lax.axis_index("s") * _NC + lax.axis_index("c")
    base = wid * _CHUNK
    pltpu.sync_copy(loss_hbm.at[pl.ds(base, _CHUNK)], data_v)

    thr = jnp.float32(THRESH)

    def body(i, carry):
        sacc, cacc = carry
        for u in range(4):
            v = data_v[pl.ds((i * 4 + u) * _L, _L)]
            hard = v > thr
            sacc = sacc + jnp.where(hard, v, jnp.float32(0.0))
            cacc = cacc + jnp.where(hard, jnp.float32(1.0), jnp.float32(0.0))
        return sacc, cacc

    sacc, cacc = lax.fori_loop(
        0, _CHUNK // (_L * 4), body,
        (jnp.zeros((_L,), jnp.float32), jnp.zeros((_L,), jnp.float32)),
    )
    sum_v[...] = sacc
    cnt_v[...] = cacc
    pltpu.sync_copy(sum_v, sum_out.at[wid])
    pltpu.sync_copy(cnt_v, cnt_out.at[wid])


@functools.lru_cache
def _get_hard_stats_kernel():
    mesh = plsc.VectorSubcoreMesh(
        core_axis_name="c", subcore_axis_name="s",
        num_cores=_NC, num_subcores=_NS)
    return functools.partial(
        pl.kernel,
        mesh=mesh,
        out_type=[
            jax.ShapeDtypeStruct((_NW, _L), jnp.float32),
            jax.ShapeDtypeStruct((_NW, _L), jnp.float32),
        ],
        scratch_types=[
            pltpu.VMEM((_CHUNK,), jnp.float32),
            pltpu.VMEM((_L,), jnp.float32),
            pltpu.VMEM((_L,), jnp.float32),
        ],
        compiler_params=pltpu.CompilerParams(needs_layout_passes=False),
    )(_sc_hard_stats_body)


@functools.lru_cache
def _make_sc_hist(shift, mask, nbins):
    mesh = plsc.VectorSubcoreMesh(
        core_axis_name="c", subcore_axis_name="s",
        num_cores=_NC, num_subcores=_NS)

    @functools.partial(
        pl.kernel,
        mesh=mesh,
        out_type=[
            jax.ShapeDtypeStruct((_NW, nbins), jnp.int32),
            jax.ShapeDtypeStruct((_NW, nbins), jnp.float32),
        ],
        scratch_types=[
            pltpu.VMEM((_CHUNK,), jnp.float32),
            pltpu.VMEM((_L,), jnp.int32),
            pltpu.VMEM((_L * nbins,), jnp.int32),
            pltpu.VMEM((_L * nbins,), jnp.float32),
            pltpu.VMEM((nbins,), jnp.int32),
            pltpu.VMEM((nbins,), jnp.float32),
        ],
        compiler_params=pltpu.CompilerParams(needs_layout_passes=False),
    )
    def _sc_hist(loss_hbm, prefix_hbm, cnt_out, sum_out,
                 data_v, pref_v, histc_v, hists_v, outc_v, outs_v):
        wid = lax.axis_index("s") * _NC + lax.axis_index("c")
        base = wid * _CHUNK
        pltpu.sync_copy(loss_hbm.at[pl.ds(base, _CHUNK)], data_v)
        pltpu.sync_copy(prefix_hbm, pref_v)

        zi = jnp.zeros((_L,), jnp.int32)
        zf = jnp.zeros((_L,), jnp.float32)

        def zero_body(i, _):
            histc_v[pl.ds(i * _L, _L)] = zi
            hists_v[pl.ds(i * _L, _L)] = zf
            return 0

        lax.fori_loop(0, nbins, zero_body, 0)

        pref = pref_v[...]
        lane_off = lax.iota(jnp.int32, _L) * nbins
        ones = jnp.ones((_L,), jnp.int32)
        maskc = _i32(mask)
        binm = _i32(nbins - 1)

        def body(i, _):
            v = data_v[pl.ds(i * _L, _L)]
            bits = lax.bitcast_convert_type(v, jnp.int32)
            match = (bits & maskc) == pref
            bn = lax.shift_right_logical(bits, shift) & binm
            idx = lane_off + bn
            plsc.addupdate_scatter(histc_v, [idx], ones, mask=match)
            plsc.addupdate_scatter(hists_v, [idx], v, mask=match)
            return 0

        lax.fori_loop(0, _CHUNK // _L, body, 0)

        def red_body(g, _):
            accc = zi
            accs = zf
            for l in range(_L):
                accc = accc + histc_v[pl.ds(l * nbins + g * _L, _L)]
                accs = accs + hists_v[pl.ds(l * nbins + g * _L, _L)]
            outc_v[pl.ds(g * _L, _L)] = accc
            outs_v[pl.ds(g * _L, _L)] = accs
            return 0

        lax.fori_loop(0, nbins // _L, red_body, 0)
        pltpu.sync_copy(outc_v, cnt_out.at[wid])
        pltpu.sync_copy(outs_v, sum_out.at[wid])

    return _sc_hist


def _select_level(counts, sums, k_rem):
    cum_ge_c = jnp.cumsum(counts[::-1])[::-1]
    cum_ge_s = jnp.cumsum(sums[::-1])[::-1]
    idx = jnp.arange(counts.shape[0], dtype=jnp.int32)
    b = jnp.max(jnp.where(cum_ge_c >= k_rem, idx, -1))
    cnt_above = cum_ge_c[b] - counts[b]
    sum_above = cum_ge_s[b] - sums[b]
    return b, cnt_above, sum_above, k_rem - cnt_above


def _topk_mean(loss_flat, k):
    kf = k.astype(jnp.float32)

    def level(hist_fn, prefix_bits):
        pref = jnp.broadcast_to(prefix_bits.astype(jnp.int32), (_L,))
        cnt_p, sum_p = hist_fn(loss_flat, pref)
        return cnt_p.sum(axis=0), sum_p.sum(axis=0)

    c1, s1 = level(_make_sc_hist(21, 0x00000000, 2048), jnp.int32(0))
    b1, ca1, sa1, k1 = _select_level(c1, s1, k)
    pref2 = lax.shift_left(b1, 21)
    c2, s2 = level(_make_sc_hist(10, 0xFFE00000, 2048), pref2)
    b2, ca2, sa2, k2 = _select_level(c2, s2, k1)
    pref3 = pref2 | lax.shift_left(b2, 10)
    c3, s3 = level(_make_sc_hist(0, 0xFFFFFC00, 1024), pref3)
    b3, ca3, sa3, k3 = _select_level(c3, s3, k2)
    v_bits = pref3 | b3
    v = lax.bitcast_convert_type(v_bits, jnp.float32)
    count_gt = ca1 + ca2 + ca3
    sum_gt = sa1 + sa2 + sa3
    return (sum_gt + (k - count_gt).astype(jnp.float32) * v) / kf


def kernel(pred, targets):
    loss_flat = _loss_pallas(pred, targets).reshape(N)
    sum_p, cnt_p = _get_hard_stats_kernel()(loss_flat)
    sum_hard = jnp.sum(sum_p)
    count_hard = jnp.sum(cnt_p)
    return lax.cond(
        count_hard >= N_MIN,
        lambda args: args[1] / args[2],
        lambda args: _topk_mean(args[0], jnp.int32(N_MIN)),
        (loss_flat, sum_hard, count_hard),
    )


# speedup vs baseline: 3.7053x; 1.0527x over previous
import functools

import numpy as np
import jax
import jax.numpy as jnp
from jax import lax
from jax.experimental import pallas as pl
from jax.experimental.pallas import tpu as pltpu
from jax.experimental.pallas import tpu_sc as plsc

B, C, H, W = 4, 19, 512, 512
N = B * H * W
N_MIN = N // 16
THRESH = float(-np.log(0.7))

BH = 128

_NC, _NS, _L = 2, 16, 16
_NW = _NC * _NS
_CHUNK = N // _NW


def _i32(x):
    return jnp.int32(np.int32(np.uint32(x)))


def _loss_body(pred_ref, tgt_ref, loss_ref):
    m = pred_ref[0, 0]
    for c in range(1, C):
        m = jnp.maximum(m, pred_ref[0, c])
    s = jnp.exp(pred_ref[0, 0] - m)
    for c in range(1, C):
        s = s + jnp.exp(pred_ref[0, c] - m)
    tb = tgt_ref[0, 0]
    pb = pred_ref[0, 0]
    for c in range(1, C):
        tc = tgt_ref[0, c]
        upd = tc > tb
        tb = jnp.where(upd, tc, tb)
        pb = jnp.where(upd, pred_ref[0, c], pb)
    loss_ref[0] = jnp.maximum((m + jnp.log(s)) - pb, 0.0)


def _loss_pallas(pred, targets):
    return pl.pallas_call(
        _loss_body,
        grid=(B, H // BH),
        in_specs=[
            pl.BlockSpec((1, C, BH, W), lambda b, i: (b, 0, i, 0)),
            pl.BlockSpec((1, C, BH, W), lambda b, i: (b, 0, i, 0)),
        ],
        out_specs=pl.BlockSpec((1, BH, W), lambda b, i: (b, i, 0)),
        out_shape=jax.ShapeDtypeStruct((B, H, W), jnp.float32),
    )(pred, targets)


def _sc_hard_stats_body(loss_hbm, sum_out, cnt_out, data_v, sum_v, cnt_v):
    wid = lax.axis_index("s") * _NC + lax.axis_index("c")
    base = wid * _CHUNK
    pltpu.sync_copy(loss_hbm.at[pl.ds(base, _CHUNK)], data_v)

    thr = jnp.float32(THRESH)

    def body(i, carry):
        sacc, cacc = carry
        for u in range(4):
            v = data_v[pl.ds((i * 4 + u) * _L, _L)]
            hard = v > thr
            sacc = sacc + jnp.where(hard, v, jnp.float32(0.0))
            cacc = cacc + jnp.where(hard, jnp.float32(1.0), jnp.float32(0.0))
        return sacc, cacc

    sacc, cacc = lax.fori_loop(
        0, _CHUNK // (_L * 4), body,
        (jnp.zeros((_L,), jnp.float32), jnp.zeros((_L,), jnp.float32)),
    )
    sum_v[...] = sacc
    cnt_v[...] = cacc
    pltpu.sync_copy(sum_v, sum_out.at[wid])
    pltpu.sync_copy(cnt_v, cnt_out.at[wid])


@functools.lru_cache
def _get_hard_stats_kernel():
    mesh = plsc.VectorSubcoreMesh(
        core_axis_name="c", subcore_axis_name="s",
        num_cores=_NC, num_subcores=_NS)
    return functools.partial(
        pl.kernel,
        mesh=mesh,
        out_type=[
            jax.ShapeDtypeStruct((_NW, _L), jnp.float32),
            jax.ShapeDtypeStruct((_NW, _L), jnp.float32),
        ],
        scratch_types=[
            pltpu.VMEM((_CHUNK,), jnp.float32),
            pltpu.VMEM((_L,), jnp.float32),
            pltpu.VMEM((_L,), jnp.float32),
        ],
        compiler_params=pltpu.CompilerParams(needs_layout_passes=False),
    )(_sc_hard_stats_body)


@functools.lru_cache
def _make_sc_hist(shift, mask, nbins):
    mesh = plsc.VectorSubcoreMesh(
        core_axis_name="c", subcore_axis_name="s",
        num_cores=_NC, num_subcores=_NS)

    @functools.partial(
        pl.kernel,
        mesh=mesh,
        out_type=[
            jax.ShapeDtypeStruct((_NW, nbins), jnp.int32),
            jax.ShapeDtypeStruct((_NW, nbins), jnp.float32),
        ],
        scratch_types=[
            pltpu.VMEM((_CHUNK,), jnp.float32),
            pltpu.VMEM((_L,), jnp.int32),
            pltpu.VMEM((_L * nbins,), jnp.int32),
            pltpu.VMEM((_L * nbins,), jnp.float32),
            pltpu.VMEM((nbins,), jnp.int32),
            pltpu.VMEM((nbins,), jnp.float32),
        ],
        compiler_params=pltpu.CompilerParams(needs_layout_passes=False),
    )
    def _sc_hist(loss_hbm, prefix_hbm, cnt_out, sum_out,
                 data_v, pref_v, histc_v, hists_v, outc_v, outs_v):
        wid = lax.axis_index("s") * _NC + lax.axis_index("c")
        base = wid * _CHUNK
        pltpu.sync_copy(loss_hbm.at[pl.ds(base, _CHUNK)], data_v)
        pltpu.sync_copy(prefix_hbm, pref_v)

        zi = jnp.zeros((_L,), jnp.int32)
        zf = jnp.zeros((_L,), jnp.float32)

        def zero_body(i, _):
            histc_v[pl.ds(i * _L, _L)] = zi
            hists_v[pl.ds(i * _L, _L)] = zf
            return 0

        lax.fori_loop(0, nbins, zero_body, 0)

        pref = pref_v[...]
        lane_off = lax.iota(jnp.int32, _L) * nbins
        ones = jnp.ones((_L,), jnp.int32)
        maskc = _i32(mask)
        binm = _i32(nbins - 1)

        def body(i, _):
            v = data_v[pl.ds(i * _L, _L)]
            bits = lax.bitcast_convert_type(v, jnp.int32)
            match = (bits & maskc) == pref
            bn = lax.shift_right_logical(bits, shift) & binm
            idx = lane_off + bn
            plsc.addupdate_scatter(histc_v, [idx], ones, mask=match)
            plsc.addupdate_scatter(hists_v, [idx], v, mask=match)
            return 0

        lax.fori_loop(0, _CHUNK // _L, body, 0)

        def red_body(g, _):
            accc = zi
            accs = zf
            for l in range(_L):
                accc = accc + histc_v[pl.ds(l * nbins + g * _L, _L)]
                accs = accs + hists_v[pl.ds(l * nbins + g * _L, _L)]
            outc_v[pl.ds(g * _L, _L)] = accc
            outs_v[pl.ds(g * _L, _L)] = accs
            return 0

        lax.fori_loop(0, nbins // _L, red_body, 0)
        pltpu.sync_copy(outc_v, cnt_out.at[wid])
        pltpu.sync_copy(outs_v, sum_out.at[wid])

    return _sc_hist


def _select_level(counts, sums, k_rem):
    cum_ge_c = jnp.cumsum(counts[::-1])[::-1]
    cum_ge_s = jnp.cumsum(sums[::-1])[::-1]
    idx = jnp.arange(counts.shape[0], dtype=jnp.int32)
    b = jnp.max(jnp.where(cum_ge_c >= k_rem, idx, -1))
    cnt_above = cum_ge_c[b] - counts[b]
    sum_above = cum_ge_s[b] - sums[b]
    return b, cnt_above, sum_above, k_rem - cnt_above


def _topk_mean(loss_flat, k):
    kf = k.astype(jnp.float32)

    def level(hist_fn, prefix_bits):
        pref = jnp.broadcast_to(prefix_bits.astype(jnp.int32), (_L,))
        cnt_p, sum_p = hist_fn(loss_flat, pref)
        return cnt_p.sum(axis=0), sum_p.sum(axis=0)

    c1, s1 = level(_make_sc_hist(21, 0x00000000, 2048), jnp.int32(0))
    b1, ca1, sa1, k1 = _select_level(c1, s1, k)
    pref2 = lax.shift_left(b1, 21)
    c2, s2 = level(_make_sc_hist(10, 0xFFE00000, 2048), pref2)
    b2, ca2, sa2, k2 = _select_level(c2, s2, k1)
    pref3 = pref2 | lax.shift_left(b2, 10)
    c3, s3 = level(_make_sc_hist(0, 0xFFFFFC00, 1024), pref3)
    b3, ca3, sa3, k3 = _select_level(c3, s3, k2)
    v_bits = pref3 | b3
    v = lax.bitcast_convert_type(v_bits, jnp.float32)
    count_gt = ca1 + ca2 + ca3
    sum_gt = sa1 + sa2 + sa3
    return (sum_gt + (k - count_gt).astype(jnp.float32) * v) / kf


def kernel(pred, targets):
    loss_flat = _loss_pallas(pred, targets).reshape(N)
    sum_p, cnt_p = _get_hard_stats_kernel()(loss_flat)
    sum_hard = jnp.sum(sum_p)
    count_hard = jnp.sum(cnt_p)
    return lax.cond(
        count_hard >= N_MIN,
        lambda args: args[1] / args[2],
        lambda args: _topk_mean(args[0], jnp.int32(N_MIN)),
        (loss_flat, sum_hard, count_hard),
    )
